# baseline (speedup 1.0000x reference)
import functools

import jax
import jax.numpy as jnp
from jax import lax
from jax.experimental import pallas as pl
from jax.experimental.pallas import tpu as pltpu
from jax.experimental.pallas import tpu_sc as plsc


def _tc_body(L, LB, H, hs_ref, wk_ref, bk_ref, wv_ref, bv_ref, rhm_ref,
             v_ref, r0_ref, r1_ref, c0_ref, c1_ref):
    b = pl.program_id(0)
    lb = pl.program_id(1)
    hs = hs_ref[...]
    k = jnp.dot(hs, wk_ref[...], preferred_element_type=jnp.float32)
    k = jnp.maximum(k + bk_ref[...], 0.0)
    hs_hi = hs.astype(jnp.bfloat16)
    v = jnp.dot(hs_hi, wv_ref[...].astype(jnp.bfloat16),
                preferred_element_type=jnp.float32)
    v = jnp.maximum(v + bv_ref[...], 0.0)
    v_ref[...] = v.astype(jnp.bfloat16)

    dotp = jnp.dot(k, rhm_ref[...], preferred_element_type=jnp.float32)
    mask = dotp > 0.5

    liota = lb * LB + lax.broadcasted_iota(jnp.int32, (LB, H), 0)
    neg = jnp.full((LB, H), -1, jnp.int32)
    s0 = jnp.where(mask, liota, neg)
    s1 = neg

    d = 1
    while d < LB:
        p0 = jnp.concatenate([neg[:d], s0[: LB - d]], axis=0)
        p1 = jnp.concatenate([neg[:d], s1[: LB - d]], axis=0)
        n0 = jnp.where(s0 >= 0, s0, p0)
        n1 = jnp.where(s0 >= 0, jnp.where(s1 >= 0, s1, p0), p1)
        s0, s1 = n0, n1
        d *= 2

    @pl.when(lb == 0)
    def _reset():
        c0_ref[...] = jnp.full((1, H), -1, jnp.int32)
        c1_ref[...] = jnp.full((1, H), -1, jnp.int32)

    c0 = c0_ref[...]
    c1 = c1_ref[...]
    n0 = jnp.where(s0 >= 0, s0, c0)
    n1 = jnp.where(s0 >= 0, jnp.where(s1 >= 0, s1, c0), c1)
    c0_ref[...] = n0[LB - 1:LB, :]
    c1_ref[...] = n1[LB - 1:LB, :]

    l0 = jnp.maximum(n0, 0)
    l1 = jnp.maximum(n1, 0)
    hiota = lax.broadcasted_iota(jnp.int32, (LB, H), 1)
    base = b * (L * H)
    r0_ref[...] = base + l0 * H + hiota
    r1_ref[...] = base + l1 * H + hiota


def _tc_call(hs2, Wk, bk2, Wv, bv2, rhm, B, L, D, AH, H):
    LB = 1024
    nlb = L // LB
    return pl.pallas_call(
        functools.partial(_tc_body, L, LB, H),
        grid=(B, nlb),
        in_specs=[
            pl.BlockSpec((LB, D), lambda b, lb: (b * (L // LB) + lb, 0)),
            pl.BlockSpec((D, AH), lambda b, lb: (0, 0)),
            pl.BlockSpec((1, AH), lambda b, lb: (0, 0)),
            pl.BlockSpec((D, AH), lambda b, lb: (0, 0)),
            pl.BlockSpec((1, AH), lambda b, lb: (0, 0)),
            pl.BlockSpec((AH, H), lambda b, lb: (0, 0)),
        ],
        out_specs=[
            pl.BlockSpec((LB, AH), lambda b, lb: (b * (L // LB) + lb, 0)),
            pl.BlockSpec((LB, H), lambda b, lb: (b * (L // LB) + lb, 0)),
            pl.BlockSpec((LB, H), lambda b, lb: (b * (L // LB) + lb, 0)),
        ],
        out_shape=[
            jax.ShapeDtypeStruct((B * L, AH), jnp.bfloat16),
            jax.ShapeDtypeStruct((B * L, H), jnp.int32),
            jax.ShapeDtypeStruct((B * L, H), jnp.int32),
        ],
        scratch_shapes=[
            pltpu.VMEM((1, H), jnp.int32),
            pltpu.VMEM((1, H), jnp.int32),
        ],
    )(hs2, Wk, bk2, Wv, bv2, rhm)


_CHUNK = 128


def _sc_call(table, r0m, r1m, w1v, w2v, rows, hd):
    info = plsc.get_sparse_core_info()
    nw = info.num_cores * info.num_subcores
    per_w = rows // nw
    n_chunks = per_w // _CHUNK
    mesh = plsc.VectorSubcoreMesh(core_axis_name="c", subcore_axis_name="s")

    @functools.partial(
        pl.kernel,
        mesh=mesh,
        compiler_params=pltpu.CompilerParams(use_tc_tiling_on_sc=False),
        out_type=jax.ShapeDtypeStruct((rows, hd), jnp.bfloat16),
        scratch_types=[
            pltpu.VMEM((n_chunks, _CHUNK), jnp.int32),
            pltpu.VMEM((n_chunks, _CHUNK), jnp.int32),
            pltpu.VMEM((2, _CHUNK, hd), jnp.bfloat16),
            pltpu.VMEM((2, _CHUNK, hd), jnp.bfloat16),
            pltpu.VMEM((2, _CHUNK, hd), jnp.bfloat16),
            pltpu.VMEM((32,), jnp.bfloat16),
            pltpu.VMEM((32,), jnp.bfloat16),
            pltpu.SemaphoreType.DMA,
            pltpu.SemaphoreType.DMA,
            pltpu.SemaphoreType.DMA,
            pltpu.SemaphoreType.DMA,
        ],
    )
    def sc_kernel(table_hbm, r0_hbm, r1_hbm, w1_hbm, w2_hbm, out_hbm,
                  idx0, idx1, buf_a, buf_b, buf_o, w1s, w2s,
                  sg0, sg1, sw0, sw1):
        wid = lax.axis_index("s") * info.num_cores + lax.axis_index("c")
        ibase = wid * n_chunks
        pltpu.sync_copy(r0_hbm.at[pl.ds(ibase, n_chunks)], idx0)
        pltpu.sync_copy(r1_hbm.at[pl.ds(ibase, n_chunks)], idx1)
        pltpu.sync_copy(w1_hbm, w1s)
        pltpu.sync_copy(w2_hbm, w2s)
        w1x = w1s[...]
        w2x = w2s[...]

        sem_g = (sg0, sg1)
        sem_w = (sw0, sw1)
        gathers = {}
        writes = {}

        def start_gather(c):
            s = c & 1
            a = pltpu.async_copy(table_hbm.at[idx0.at[c]], buf_a.at[s],
                                 sem_g[s])
            b = pltpu.async_copy(table_hbm.at[idx1.at[c]], buf_b.at[s],
                                 sem_g[s])
            gathers[c] = (a, b)

        start_gather(0)
        for c in range(n_chunks):
            if c + 1 < n_chunks:
                start_gather(c + 1)
            ga, gb = gathers.pop(c)
            ga.wait()
            gb.wait()
            if c >= 2:
                writes.pop(c - 2).wait()
            s = c & 1

            def row(i, carry, s=s):
                for j in range(hd // 32):
                    a = buf_a[s, i, pl.ds(j * 32, 32)]
                    bb = buf_b[s, i, pl.ds(j * 32, 32)]
                    buf_o[s, i, pl.ds(j * 32, 32)] = a * w1x + bb * w2x
                return carry

            lax.fori_loop(0, _CHUNK, row, 0)
            writes[c] = pltpu.async_copy(
                buf_o.at[s], out_hbm.at[pl.ds(wid * per_w + c * _CHUNK,
                                              _CHUNK)], sem_w[s])
        writes.pop(n_chunks - 2).wait()
        writes.pop(n_chunks - 1).wait()

    return sc_kernel(table, r0m, r1m, w1v, w2v)


def kernel(hidden_states, Wk, bk, Wv, bv, w1, w2, ReadingHead):
    B, L, D = hidden_states.shape
    H, HD = ReadingHead.shape
    AH = H * HD

    hs2 = hidden_states.reshape(B * L, D)
    bk2 = bk.reshape(1, AH)
    bv2 = bv.reshape(1, AH)
    rhm = (ReadingHead[:, :, None]
           * jnp.eye(H, dtype=ReadingHead.dtype)[:, None, :]).reshape(AH, H)

    v1, r0, r1 = _tc_call(hs2, Wk, bk2, Wv, bv2, rhm, B, L, D, AH, H)

    table = v1.reshape(B * L * H, HD)
    r0m = r0.reshape(B * L * H // _CHUNK, _CHUNK)
    r1m = r1.reshape(B * L * H // _CHUNK, _CHUNK)
    w1v = jnp.broadcast_to(w1.reshape(()), (32,)).astype(jnp.bfloat16)
    w2v = jnp.broadcast_to(w2.reshape(()), (32,)).astype(jnp.bfloat16)

    out = _sc_call(table, r0m, r1m, w1v, w2v, B * L * H, HD)
    return (out.astype(jnp.float32).reshape(B, L, AH),)


# speedup vs baseline: 1.5331x; 1.5331x over previous
import functools

import jax
import jax.numpy as jnp
from jax import lax
from jax.experimental import pallas as pl
from jax.experimental.pallas import tpu as pltpu
from jax.experimental.pallas import tpu_sc as plsc


def _tc_body(L, LB, H, hs_ref, wk_ref, bk_ref, wv_ref, bv_ref, rhm_ref,
             v_ref, r0_ref, r1_ref, c0_ref, c1_ref):
    b = pl.program_id(0)
    lb = pl.program_id(1)
    hs = hs_ref[...]
    k = jnp.dot(hs, wk_ref[...], preferred_element_type=jnp.float32)
    k = jnp.maximum(k + bk_ref[...], 0.0)
    hs_hi = hs.astype(jnp.bfloat16)
    v = jnp.dot(hs_hi, wv_ref[...].astype(jnp.bfloat16),
                preferred_element_type=jnp.float32)
    v = jnp.maximum(v + bv_ref[...], 0.0)
    v_ref[...] = v

    dotp = jnp.dot(k, rhm_ref[...], preferred_element_type=jnp.float32)
    mask = dotp > 0.5

    liota = lb * LB + lax.broadcasted_iota(jnp.int32, (LB, H), 0)
    neg = jnp.full((LB, H), -1, jnp.int32)
    s0 = jnp.where(mask, liota, neg)
    s1 = neg

    d = 1
    while d < LB:
        p0 = jnp.concatenate([neg[:d], s0[: LB - d]], axis=0)
        p1 = jnp.concatenate([neg[:d], s1[: LB - d]], axis=0)
        n0 = jnp.where(s0 >= 0, s0, p0)
        n1 = jnp.where(s0 >= 0, jnp.where(s1 >= 0, s1, p0), p1)
        s0, s1 = n0, n1
        d *= 2

    @pl.when(lb == 0)
    def _reset():
        c0_ref[...] = jnp.full((1, H), -1, jnp.int32)
        c1_ref[...] = jnp.full((1, H), -1, jnp.int32)

    c0 = c0_ref[...]
    c1 = c1_ref[...]
    n0 = jnp.where(s0 >= 0, s0, c0)
    n1 = jnp.where(s0 >= 0, jnp.where(s1 >= 0, s1, c0), c1)
    c0_ref[...] = n0[LB - 1:LB, :]
    c1_ref[...] = n1[LB - 1:LB, :]

    l0 = jnp.maximum(n0, 0)
    l1 = jnp.maximum(n1, 0)
    hiota = lax.broadcasted_iota(jnp.int32, (LB, H), 1)
    base = b * (L * H)
    r0_ref[...] = base + l0 * H + hiota
    r1_ref[...] = base + l1 * H + hiota


def _tc_call(hs2, Wk, bk2, Wv, bv2, rhm, B, L, D, AH, H):
    LB = 1024
    nlb = L // LB
    return pl.pallas_call(
        functools.partial(_tc_body, L, LB, H),
        grid=(B, nlb),
        in_specs=[
            pl.BlockSpec((LB, D), lambda b, lb: (b * (L // LB) + lb, 0)),
            pl.BlockSpec((D, AH), lambda b, lb: (0, 0)),
            pl.BlockSpec((1, AH), lambda b, lb: (0, 0)),
            pl.BlockSpec((D, AH), lambda b, lb: (0, 0)),
            pl.BlockSpec((1, AH), lambda b, lb: (0, 0)),
            pl.BlockSpec((AH, H), lambda b, lb: (0, 0)),
        ],
        out_specs=[
            pl.BlockSpec((LB, AH), lambda b, lb: (b * (L // LB) + lb, 0)),
            pl.BlockSpec((LB, H), lambda b, lb: (b * (L // LB) + lb, 0)),
            pl.BlockSpec((LB, H), lambda b, lb: (b * (L // LB) + lb, 0)),
        ],
        out_shape=[
            jax.ShapeDtypeStruct((B * L, AH), jnp.float32),
            jax.ShapeDtypeStruct((B * L, H), jnp.int32),
            jax.ShapeDtypeStruct((B * L, H), jnp.int32),
        ],
        scratch_shapes=[
            pltpu.VMEM((1, H), jnp.int32),
            pltpu.VMEM((1, H), jnp.int32),
        ],
    )(hs2, Wk, bk2, Wv, bv2, rhm)


_CHUNK = 128


def _sc_call(table, r0m, r1m, w1v, w2v, rows, hd):
    info = plsc.get_sparse_core_info()
    nw = info.num_cores * info.num_subcores
    per_w = rows // nw
    n_chunks = per_w // _CHUNK
    mesh = plsc.VectorSubcoreMesh(core_axis_name="c", subcore_axis_name="s")

    @functools.partial(
        pl.kernel,
        mesh=mesh,
        compiler_params=pltpu.CompilerParams(use_tc_tiling_on_sc=False),
        out_type=jax.ShapeDtypeStruct((rows, hd), jnp.float32),
        scratch_types=[
            pltpu.VMEM((n_chunks, _CHUNK), jnp.int32),
            pltpu.VMEM((n_chunks, _CHUNK), jnp.int32),
            pltpu.VMEM((2, _CHUNK, hd), jnp.float32),
            pltpu.VMEM((2, _CHUNK, hd), jnp.float32),
            pltpu.VMEM((2, _CHUNK, hd), jnp.float32),
            pltpu.VMEM((16,), jnp.float32),
            pltpu.VMEM((16,), jnp.float32),
            pltpu.SemaphoreType.DMA,
            pltpu.SemaphoreType.DMA,
            pltpu.SemaphoreType.DMA,
            pltpu.SemaphoreType.DMA,
        ],
    )
    def sc_kernel(table_hbm, r0_hbm, r1_hbm, w1_hbm, w2_hbm, out_hbm,
                  idx0, idx1, buf_a, buf_b, buf_o, w1s, w2s,
                  sg0, sg1, sw0, sw1):
        wid = lax.axis_index("s") * info.num_cores + lax.axis_index("c")
        ibase = wid * n_chunks
        pltpu.sync_copy(r0_hbm.at[pl.ds(ibase, n_chunks)], idx0)
        pltpu.sync_copy(r1_hbm.at[pl.ds(ibase, n_chunks)], idx1)
        pltpu.sync_copy(w1_hbm, w1s)
        pltpu.sync_copy(w2_hbm, w2s)
        w1x = w1s[...]
        w2x = w2s[...]

        sem_g = (sg0, sg1)
        sem_w = (sw0, sw1)
        gathers = {}
        writes = {}

        def start_gather(c):
            s = c & 1
            a = pltpu.async_copy(table_hbm.at[idx0.at[c]], buf_a.at[s],
                                 sem_g[s])
            b = pltpu.async_copy(table_hbm.at[idx1.at[c]], buf_b.at[s],
                                 sem_g[s])
            gathers[c] = (a, b)

        start_gather(0)
        for c in range(n_chunks):
            if c + 1 < n_chunks:
                start_gather(c + 1)
            ga, gb = gathers.pop(c)
            ga.wait()
            gb.wait()
            if c >= 2:
                writes.pop(c - 2).wait()
            s = c & 1

            def row(i, carry, s=s):
                for j in range(hd // 16):
                    a = buf_a[s, i, pl.ds(j * 16, 16)]
                    bb = buf_b[s, i, pl.ds(j * 16, 16)]
                    buf_o[s, i, pl.ds(j * 16, 16)] = a * w1x + bb * w2x
                return carry

            lax.fori_loop(0, _CHUNK, row, 0)
            writes[c] = pltpu.async_copy(
                buf_o.at[s], out_hbm.at[pl.ds(wid * per_w + c * _CHUNK,
                                              _CHUNK)], sem_w[s])
        writes.pop(n_chunks - 2).wait()
        writes.pop(n_chunks - 1).wait()

    return sc_kernel(table, r0m, r1m, w1v, w2v)


def kernel(hidden_states, Wk, bk, Wv, bv, w1, w2, ReadingHead):
    B, L, D = hidden_states.shape
    H, HD = ReadingHead.shape
    AH = H * HD

    hs2 = hidden_states.reshape(B * L, D)
    bk2 = bk.reshape(1, AH)
    bv2 = bv.reshape(1, AH)
    rhm = (ReadingHead[:, :, None]
           * jnp.eye(H, dtype=ReadingHead.dtype)[:, None, :]).reshape(AH, H)

    v1, r0, r1 = _tc_call(hs2, Wk, bk2, Wv, bv2, rhm, B, L, D, AH, H)

    table = v1.reshape(B * L * H, HD)
    r0m = r0.reshape(B * L * H // _CHUNK, _CHUNK)
    r1m = r1.reshape(B * L * H // _CHUNK, _CHUNK)
    w1v = jnp.broadcast_to(w1.reshape(()), (16,)).astype(jnp.float32)
    w2v = jnp.broadcast_to(w2.reshape(()), (16,)).astype(jnp.float32)

    out = _sc_call(table, r0m, r1m, w1v, w2v, B * L * H, HD)
    return (out.reshape(B, L, AH),)


# speedup vs baseline: 1.5436x; 1.0069x over previous
import functools

import jax
import jax.numpy as jnp
from jax import lax
from jax.experimental import pallas as pl
from jax.experimental.pallas import tpu as pltpu
from jax.experimental.pallas import tpu_sc as plsc


def _tc_body(L, LB, H, hs_ref, wk_ref, bk_ref, wv_ref, bv_ref, rhm_ref,
             v_ref, r0_ref, r1_ref, c0_ref, c1_ref):
    b = pl.program_id(0)
    lb = pl.program_id(1)
    hs = hs_ref[...]
    k = jnp.dot(hs, wk_ref[...], preferred_element_type=jnp.float32)
    k = jnp.maximum(k + bk_ref[...], 0.0)
    hs_hi = hs.astype(jnp.bfloat16)
    v = jnp.dot(hs_hi, wv_ref[...].astype(jnp.bfloat16),
                preferred_element_type=jnp.float32)
    v = jnp.maximum(v + bv_ref[...], 0.0)
    hd = v.shape[1] // H
    vu = lax.bitcast_convert_type(v, jnp.uint32) + jnp.uint32(0x8000)
    pieces = []
    for h in range(H):
        a = vu[:, h * hd: h * hd + hd // 2]
        bhalf = vu[:, h * hd + hd // 2: (h + 1) * hd]
        pieces.append((a >> 16) | (bhalf & jnp.uint32(0xFFFF0000)))
    packed = jnp.concatenate(pieces, axis=1)
    v_ref[...] = lax.bitcast_convert_type(packed, jnp.float32)

    dotp = jnp.dot(k, rhm_ref[...], preferred_element_type=jnp.float32)
    mask = dotp > 0.5

    liota = lb * LB + lax.broadcasted_iota(jnp.int32, (LB, H), 0)
    neg = jnp.full((LB, H), -1, jnp.int32)
    s0 = jnp.where(mask, liota, neg)
    s1 = neg

    d = 1
    while d < LB:
        p0 = jnp.concatenate([neg[:d], s0[: LB - d]], axis=0)
        p1 = jnp.concatenate([neg[:d], s1[: LB - d]], axis=0)
        n0 = jnp.where(s0 >= 0, s0, p0)
        n1 = jnp.where(s0 >= 0, jnp.where(s1 >= 0, s1, p0), p1)
        s0, s1 = n0, n1
        d *= 2

    @pl.when(lb == 0)
    def _reset():
        c0_ref[...] = jnp.full((1, H), -1, jnp.int32)
        c1_ref[...] = jnp.full((1, H), -1, jnp.int32)

    c0 = c0_ref[...]
    c1 = c1_ref[...]
    n0 = jnp.where(s0 >= 0, s0, c0)
    n1 = jnp.where(s0 >= 0, jnp.where(s1 >= 0, s1, c0), c1)
    c0_ref[...] = n0[LB - 1:LB, :]
    c1_ref[...] = n1[LB - 1:LB, :]

    l0 = jnp.maximum(n0, 0)
    l1 = jnp.maximum(n1, 0)
    hiota = lax.broadcasted_iota(jnp.int32, (LB, H), 1)
    base = b * (L * H)
    r0_ref[...] = base + l0 * H + hiota
    r1_ref[...] = base + l1 * H + hiota


def _tc_call(hs2, Wk, bk2, Wv, bv2, rhm, B, L, D, AH, H):
    LB = 1024
    nlb = L // LB
    return pl.pallas_call(
        functools.partial(_tc_body, L, LB, H),
        grid=(B, nlb),
        in_specs=[
            pl.BlockSpec((LB, D), lambda b, lb: (b * (L // LB) + lb, 0)),
            pl.BlockSpec((D, AH), lambda b, lb: (0, 0)),
            pl.BlockSpec((1, AH), lambda b, lb: (0, 0)),
            pl.BlockSpec((D, AH), lambda b, lb: (0, 0)),
            pl.BlockSpec((1, AH), lambda b, lb: (0, 0)),
            pl.BlockSpec((AH, H), lambda b, lb: (0, 0)),
        ],
        out_specs=[
            pl.BlockSpec((LB, AH // 2), lambda b, lb: (b * (L // LB) + lb, 0)),
            pl.BlockSpec((LB, H), lambda b, lb: (b * (L // LB) + lb, 0)),
            pl.BlockSpec((LB, H), lambda b, lb: (b * (L // LB) + lb, 0)),
        ],
        out_shape=[
            jax.ShapeDtypeStruct((B * L, AH // 2), jnp.float32),
            jax.ShapeDtypeStruct((B * L, H), jnp.int32),
            jax.ShapeDtypeStruct((B * L, H), jnp.int32),
        ],
        scratch_shapes=[
            pltpu.VMEM((1, H), jnp.int32),
            pltpu.VMEM((1, H), jnp.int32),
        ],
    )(hs2, Wk, bk2, Wv, bv2, rhm)


_CHUNK = 128


def _sc_call(table, r0m, r1m, w1v, w2v, rows, hd):
    info = plsc.get_sparse_core_info()
    nw = info.num_cores * info.num_subcores
    per_w = rows // nw
    n_chunks = per_w // _CHUNK
    mesh = plsc.VectorSubcoreMesh(core_axis_name="c", subcore_axis_name="s")

    @functools.partial(
        pl.kernel,
        mesh=mesh,
        compiler_params=pltpu.CompilerParams(use_tc_tiling_on_sc=False),
        out_type=jax.ShapeDtypeStruct((rows, hd), jnp.float32),
        scratch_types=[
            pltpu.VMEM((n_chunks, _CHUNK), jnp.int32),
            pltpu.VMEM((n_chunks, _CHUNK), jnp.int32),
            pltpu.VMEM((2, _CHUNK, hd // 2), jnp.float32),
            pltpu.VMEM((2, _CHUNK, hd // 2), jnp.float32),
            pltpu.VMEM((2, _CHUNK, hd), jnp.float32),
            pltpu.VMEM((16,), jnp.float32),
            pltpu.VMEM((16,), jnp.float32),
            pltpu.SemaphoreType.DMA,
            pltpu.SemaphoreType.DMA,
            pltpu.SemaphoreType.DMA,
            pltpu.SemaphoreType.DMA,
        ],
    )
    def sc_kernel(table_hbm, r0_hbm, r1_hbm, w1_hbm, w2_hbm, out_hbm,
                  idx0, idx1, buf_a, buf_b, buf_o, w1s, w2s,
                  sg0, sg1, sw0, sw1):
        wid = lax.axis_index("s") * info.num_cores + lax.axis_index("c")
        ibase = wid * n_chunks
        pltpu.sync_copy(r0_hbm.at[pl.ds(ibase, n_chunks)], idx0)
        pltpu.sync_copy(r1_hbm.at[pl.ds(ibase, n_chunks)], idx1)
        pltpu.sync_copy(w1_hbm, w1s)
        pltpu.sync_copy(w2_hbm, w2s)
        w1x = w1s[...]
        w2x = w2s[...]

        sem_g = (sg0, sg1)
        sem_w = (sw0, sw1)
        gathers = {}
        writes = {}

        def start_gather(c):
            s = c & 1
            a = pltpu.async_copy(table_hbm.at[idx0.at[c]], buf_a.at[s],
                                 sem_g[s])
            b = pltpu.async_copy(table_hbm.at[idx1.at[c]], buf_b.at[s],
                                 sem_g[s])
            gathers[c] = (a, b)

        start_gather(0)
        for c in range(n_chunks):
            if c + 1 < n_chunks:
                start_gather(c + 1)
            ga, gb = gathers.pop(c)
            ga.wait()
            gb.wait()
            if c >= 2:
                writes.pop(c - 2).wait()
            s = c & 1

            def row(i, carry, s=s):
                himask = jnp.uint32(0xFFFF0000)
                for j in range(hd // 32):
                    wa = lax.bitcast_convert_type(
                        buf_a[s, i, pl.ds(j * 16, 16)], jnp.uint32)
                    wb = lax.bitcast_convert_type(
                        buf_b[s, i, pl.ds(j * 16, 16)], jnp.uint32)
                    alo = lax.bitcast_convert_type(wa << 16, jnp.float32)
                    ahi = lax.bitcast_convert_type(wa & himask, jnp.float32)
                    blo = lax.bitcast_convert_type(wb << 16, jnp.float32)
                    bhi = lax.bitcast_convert_type(wb & himask, jnp.float32)
                    buf_o[s, i, pl.ds(j * 16, 16)] = alo * w1x + blo * w2x
                    buf_o[s, i, pl.ds(hd // 2 + j * 16, 16)] = (
                        ahi * w1x + bhi * w2x)
                return carry

            lax.fori_loop(0, _CHUNK, row, 0)
            writes[c] = pltpu.async_copy(
                buf_o.at[s], out_hbm.at[pl.ds(wid * per_w + c * _CHUNK,
                                              _CHUNK)], sem_w[s])
        writes.pop(n_chunks - 2).wait()
        writes.pop(n_chunks - 1).wait()

    return sc_kernel(table, r0m, r1m, w1v, w2v)


def kernel(hidden_states, Wk, bk, Wv, bv, w1, w2, ReadingHead):
    B, L, D = hidden_states.shape
    H, HD = ReadingHead.shape
    AH = H * HD

    hs2 = hidden_states.reshape(B * L, D)
    bk2 = bk.reshape(1, AH)
    bv2 = bv.reshape(1, AH)
    rhm = (ReadingHead[:, :, None]
           * jnp.eye(H, dtype=ReadingHead.dtype)[:, None, :]).reshape(AH, H)

    v1, r0, r1 = _tc_call(hs2, Wk, bk2, Wv, bv2, rhm, B, L, D, AH, H)

    table = v1.reshape(B * L * H, HD // 2)
    r0m = r0.reshape(B * L * H // _CHUNK, _CHUNK)
    r1m = r1.reshape(B * L * H // _CHUNK, _CHUNK)
    w1v = jnp.broadcast_to(w1.reshape(()), (16,)).astype(jnp.float32)
    w2v = jnp.broadcast_to(w2.reshape(()), (16,)).astype(jnp.float32)

    out = _sc_call(table, r0m, r1m, w1v, w2v, B * L * H, HD)
    return (out.reshape(B, L, AH),)
